# Initial kernel scaffold; baseline (speedup 1.0000x reference)
import jax
import jax.numpy as jnp
from jax.experimental import pallas as pl


def kernel(xyz, feature, mlp_params, conv1_W, conv1_b, bn1_gamma, bn1_beta, fps_idx):
    raise NotImplementedError("write your pallas kernel here")


# speedup vs baseline: 1.3134x; 1.3134x over previous
import functools

import jax
import jax.numpy as jnp
from jax.experimental import pallas as pl

_B, _N = 2, 8192
_NPOINTS = 2048
_RADII = [0.2, 0.4, 0.8]
_NSAMPLES = [16, 32, 64]
_SUBL, _LANE = 64, 128
_OSUB = 16


def _fps_body(xr, inds_ref, nxyz_ref):
    X = xr[0, 0]
    Y = xr[0, 1]
    Z = xr[0, 2]
    r_io = jax.lax.broadcasted_iota(jnp.int32, (_SUBL, _LANE), 0)
    c_io = jax.lax.broadcasted_iota(jnp.int32, (_SUBL, _LANE), 1)
    fi = r_io * _LANE + c_io
    r2 = jax.lax.broadcasted_iota(jnp.int32, (_OSUB, _LANE), 0)
    c2 = jax.lax.broadcasted_iota(jnp.int32, (_OSUB, _LANE), 1)
    fo = r2 * _LANE + c2
    BIG = jnp.int32(1 << 30)

    def step(i, st):
        dist, far, idxs, nx, ny, nz = st
        sel = fi == far
        cx = jnp.sum(jnp.where(sel, X, 0.0))
        cy = jnp.sum(jnp.where(sel, Y, 0.0))
        cz = jnp.sum(jnp.where(sel, Z, 0.0))
        dx = X - cx
        dy = Y - cy
        dz = Z - cz
        d = (dx * dx + dy * dy) + dz * dz
        dist = jnp.minimum(dist, d)
        m = jnp.max(dist)
        far_new = jnp.min(jnp.where(dist == m, fi, BIG))
        w = fo == i
        idxs = jnp.where(w, far, idxs)
        nx = jnp.where(w, cx, nx)
        ny = jnp.where(w, cy, ny)
        nz = jnp.where(w, cz, nz)
        return (dist, far_new, idxs, nx, ny, nz)

    dist0 = jnp.full((_SUBL, _LANE), 1e10, jnp.float32)
    zi = jnp.zeros((_OSUB, _LANE), jnp.int32)
    zf = jnp.zeros((_OSUB, _LANE), jnp.float32)
    _, _, idxs, nx, ny, nz = jax.lax.fori_loop(
        0, _NPOINTS, step, (dist0, jnp.int32(0), zi, zf, zf, zf))
    inds_ref[0] = idxs
    nxyz_ref[0, 0] = nx
    nxyz_ref[0, 1] = ny
    nxyz_ref[0, 2] = nz


def _run_fps(xyz):
    xr = xyz.transpose(0, 2, 1).reshape(_B, 3, _SUBL, _LANE)
    inds, nxyz = pl.pallas_call(
        _fps_body,
        grid=(_B,),
        in_specs=[pl.BlockSpec((1, 3, _SUBL, _LANE), lambda b: (b, 0, 0, 0))],
        out_specs=[
            pl.BlockSpec((1, _OSUB, _LANE), lambda b: (b, 0, 0)),
            pl.BlockSpec((1, 3, _OSUB, _LANE), lambda b: (b, 0, 0, 0)),
        ],
        out_shape=[
            jax.ShapeDtypeStruct((_B, _OSUB, _LANE), jnp.int32),
            jax.ShapeDtypeStruct((_B, 3, _OSUB, _LANE), jnp.float32),
        ],
    )(xr)
    inds = inds.reshape(_B, _NPOINTS)
    new_xyz = nxyz.reshape(_B, 3, _NPOINTS).transpose(0, 2, 1)
    return inds, new_xyz


def _square_distance(src, dst):
    return (jnp.sum(src ** 2, -1)[:, :, None] + jnp.sum(dst ** 2, -1)[:, None, :]
            - 2.0 * jnp.einsum('bmd,bnd->bmn', src, dst))


def _ball_query(radius, nsample, xyz, new_xyz):
    b, n, _ = xyz.shape
    m = new_xyz.shape[1]
    sqr = _square_distance(new_xyz, xyz)
    gidx = jnp.broadcast_to(jnp.arange(n, dtype=jnp.int32), (b, m, n))
    gidx = jnp.where(sqr > radius ** 2, n, gidx)
    gidx = jnp.sort(gidx, axis=-1)[:, :, :nsample]
    first = gidx[:, :, :1]
    first = jnp.where(first == n, 0, first)
    gidx = jnp.where(gidx == n, first, gidx)
    return gidx


def _gather_points(points, idx):
    bsz = points.shape[0]
    bidx = jnp.arange(bsz).reshape((bsz,) + (1,) * (idx.ndim - 1))
    return points[bidx, idx]


def _batchnorm(x, gamma, beta, axes):
    mean = jnp.mean(x, axis=axes, keepdims=True)
    var = jnp.var(x, axis=axes, keepdims=True)
    xh = (x - mean) * jax.lax.rsqrt(var + 1e-5)
    shape = [1] * x.ndim
    shape[1] = x.shape[1]
    return xh * gamma.reshape(shape) + beta.reshape(shape)


def kernel(xyz, feature, mlp_params, conv1_W, conv1_b, bn1_gamma, bn1_beta, fps_idx):
    inds, new_xyz = _run_fps(xyz)
    feat_t = jnp.transpose(feature, (0, 2, 1))
    outs = []
    for i in range(len(_RADII)):
        gidx = _ball_query(_RADII[i], _NSAMPLES[i], xyz, new_xyz)
        grouped_xyz = _gather_points(xyz, gidx) - new_xyz[:, :, None, :]
        grouped_feat = _gather_points(feat_t, gidx)
        g = jnp.concatenate([grouped_xyz, grouped_feat], axis=-1)
        x = jnp.transpose(g, (0, 3, 1, 2))
        for (W, bb, gm, bt) in mlp_params[i]:
            x = jnp.einsum('oi,bims->boms', W, x) + bb[None, :, None, None]
            x = jax.nn.relu(_batchnorm(x, gm, bt, (0, 2, 3)))
        outs.append(jnp.max(x, axis=-1))
    nf = jnp.concatenate(outs, axis=1)
    nf = jnp.einsum('oi,bim->bom', conv1_W, nf) + conv1_b[None, :, None]
    nf = jax.nn.relu(_batchnorm(nf, bn1_gamma, bn1_beta, (0, 2)))
    return new_xyz, nf, inds


# speedup vs baseline: 13.5952x; 10.3509x over previous
import functools

import jax
import jax.numpy as jnp
from jax import lax
from jax.experimental import pallas as pl
from jax.experimental.pallas import tpu as pltpu
from jax.experimental.pallas import tpu_sc as plsc

_B, _N = 2, 8192
_NPOINTS = 2048
_RADII = [0.2, 0.4, 0.8]
_NSAMPLES = [16, 32, 64]
_SUBL, _LANE = 64, 128
_OSUB = 16


def _fps_body(xr, inds_ref, nxyz_ref):
    X = xr[0, 0]
    Y = xr[0, 1]
    Z = xr[0, 2]
    r_io = jax.lax.broadcasted_iota(jnp.int32, (_SUBL, _LANE), 0)
    c_io = jax.lax.broadcasted_iota(jnp.int32, (_SUBL, _LANE), 1)
    fi = r_io * _LANE + c_io
    r2 = jax.lax.broadcasted_iota(jnp.int32, (_OSUB, _LANE), 0)
    c2 = jax.lax.broadcasted_iota(jnp.int32, (_OSUB, _LANE), 1)
    fo = r2 * _LANE + c2
    BIG = jnp.int32(1 << 30)

    def step(i, st):
        dist, far, idxs, nx, ny, nz = st
        sel = fi == far
        cx = jnp.sum(jnp.where(sel, X, 0.0))
        cy = jnp.sum(jnp.where(sel, Y, 0.0))
        cz = jnp.sum(jnp.where(sel, Z, 0.0))
        dx = X - cx
        dy = Y - cy
        dz = Z - cz
        d = (dx * dx + dy * dy) + dz * dz
        dist = jnp.minimum(dist, d)
        m = jnp.max(dist)
        far_new = jnp.min(jnp.where(dist == m, fi, BIG))
        w = fo == i
        idxs = jnp.where(w, far, idxs)
        nx = jnp.where(w, cx, nx)
        ny = jnp.where(w, cy, ny)
        nz = jnp.where(w, cz, nz)
        return (dist, far_new, idxs, nx, ny, nz)

    dist0 = jnp.full((_SUBL, _LANE), 1e10, jnp.float32)
    zi = jnp.zeros((_OSUB, _LANE), jnp.int32)
    zf = jnp.zeros((_OSUB, _LANE), jnp.float32)
    _, _, idxs, nx, ny, nz = jax.lax.fori_loop(
        0, _NPOINTS, step, (dist0, jnp.int32(0), zi, zf, zf, zf))
    inds_ref[0] = idxs
    nxyz_ref[0, 0] = nx
    nxyz_ref[0, 1] = ny
    nxyz_ref[0, 2] = nz


def _run_fps(xyz):
    xr = xyz.transpose(0, 2, 1).reshape(_B, 3, _SUBL, _LANE)
    inds, nxyz = pl.pallas_call(
        _fps_body,
        grid=(_B,),
        in_specs=[pl.BlockSpec((1, 3, _SUBL, _LANE), lambda b: (b, 0, 0, 0))],
        out_specs=[
            pl.BlockSpec((1, _OSUB, _LANE), lambda b: (b, 0, 0)),
            pl.BlockSpec((1, 3, _OSUB, _LANE), lambda b: (b, 0, 0, 0)),
        ],
        out_shape=[
            jax.ShapeDtypeStruct((_B, _OSUB, _LANE), jnp.int32),
            jax.ShapeDtypeStruct((_B, 3, _OSUB, _LANE), jnp.float32),
        ],
    )(xr)
    inds = inds.reshape(_B, _NPOINTS)
    new_xyz = nxyz.reshape(_B, 3, _NPOINTS).transpose(0, 2, 1)
    return inds, new_xyz


def _sq_body(nx_ref, x_ref, o_ref):
    nx = nx_ref[0]
    x = x_ref[0]
    s1 = jnp.sum(nx ** 2, -1)[:, None]
    s2 = jnp.sum(x ** 2, -1)[None, :]
    dot = jax.lax.dot_general(nx, x, (((1,), (1,)), ((), ())),
                              preferred_element_type=jnp.float32)
    o_ref[0] = s1 + s2 - 2.0 * dot


def _run_sq(new_xyz, xyz):
    return pl.pallas_call(
        _sq_body,
        grid=(_B, 8),
        in_specs=[pl.BlockSpec((1, 256, 3), lambda b, m: (b, m, 0)),
                  pl.BlockSpec((1, _N, 3), lambda b, m: (b, 0, 0))],
        out_specs=pl.BlockSpec((1, 256, _N), lambda b, m: (b, m, 0)),
        out_shape=jax.ShapeDtypeStruct((_B, _NPOINTS, _N), jnp.float32),
    )(new_xyz, xyz)


_NCORE, _NSUB = 2, 16
_NW = _NCORE * _NSUB
_ROWS = _B * _NPOINTS
_RPW = _ROWS // _NW
_GRP = 16
_NGRP = _RPW // _GRP
_TAB_W = _N * 7
_TOT = [_ROWS * ns for ns in _NSAMPLES]
_RAD2 = [r * r for r in _RADII]


def _sc_body(tab_hbm, cen_hbm, sq_hbm, g1_hbm, g2_hbm, g3_hbm,
             tab_v, cen_v, sq_v, gb1, gb2, gb3, go1, go2, go3, cnt_s):
    wid = lax.axis_index("c") * _NSUB + lax.axis_index("s")
    base_row = wid * _RPW
    b = base_row // _NPOINTS
    pltpu.sync_copy(tab_hbm.at[pl.ds(b * _TAB_W, _TAB_W)], tab_v)
    pltpu.sync_copy(cen_hbm.at[pl.ds(base_row * 8, _RPW * 8)],
                    cen_v.at[pl.ds(0, _RPW * 8)])
    iota = lax.iota(jnp.int32, 16)
    zeros16 = jnp.zeros((16,), jnp.int32)
    gbs = (gb1, gb2, gb3)
    gos = (go1, go2, go3)
    ghs = (g1_hbm, g2_hbm, g3_hbm)

    def group_body(gr, carry):
        def row_body(rr, carry2):
            rloc = gr * _GRP + rr
            cvec = cen_v[pl.ds(rloc * 8, 16)]
            cx = cvec[0]
            cy = cvec[1]
            cz = cvec[2]
            pltpu.sync_copy(
                sq_hbm.at[pl.ds((base_row + rloc) * _N, _N)], sq_v)
            for k in range(3):
                gbs[k][pl.ds(0, 16)] = zeros16
                cnt_s[k] = jnp.int32(0)

            def chunk_body(j, carry3):
                c1 = cnt_s[0]
                c2 = cnt_s[1]
                c3 = cnt_s[2]
                live = (c1 < 16) | (c2 < 32) | (c3 < 64)

                @pl.when(live)
                def _do():
                    idx = j * 16 + iota
                    sq = sq_v[pl.ds(j * 16, 16)]
                    cs = [c1, c2, c3]
                    for k in range(3):
                        mk = sq <= _RAD2[k]
                        csum = plsc.cumsum(mk.astype(jnp.int32))
                        pos = (cs[k] - 1) + csum
                        plsc.store_scatter(gbs[k], [pos], idx, mask=mk)
                        cnt_s[k] = jnp.minimum(
                            cs[k] + jnp.max(csum), _NSAMPLES[k])
                return carry3

            lax.fori_loop(0, _N // 16, chunk_body, 0)
            cnts = (cnt_s[0], cnt_s[1], cnt_s[2])
            for k in range(3):
                ns = _NSAMPLES[k]
                first = plsc.load_gather(gbs[k], [zeros16])
                for t in range(ns // 16):
                    lane = t * 16 + iota
                    v = gbs[k][pl.ds(t * 16, 16)]
                    v = jnp.where(lane < cnts[k], v, first)
                    v7 = v * 7
                    colb = rr * ns + t * 16
                    for c in range(7):
                        val = plsc.load_gather(tab_v, [v7 + c])
                        if c == 0:
                            val = val - cx
                        elif c == 1:
                            val = val - cy
                        elif c == 2:
                            val = val - cz
                        gos[k][pl.ds(c * (_GRP * ns) + colb, 16)] = val
            return carry2
        lax.fori_loop(0, _GRP, row_body, 0)
        for k in range(3):
            gsz = _GRP * _NSAMPLES[k]
            colbase = (base_row + gr * _GRP) * _NSAMPLES[k]
            for c in range(7):
                pltpu.sync_copy(
                    gos[k].at[pl.ds(c * gsz, gsz)],
                    ghs[k].at[pl.ds(c * _TOT[k] + colbase, gsz)])
        return carry
    lax.fori_loop(0, _NGRP, group_body, 0)


_sc_grouper = functools.partial(
    pl.kernel,
    mesh=plsc.VectorSubcoreMesh(core_axis_name="c", subcore_axis_name="s"),
    compiler_params=pltpu.CompilerParams(needs_layout_passes=False),
    out_type=[
        jax.ShapeDtypeStruct((7 * _TOT[0],), jnp.float32),
        jax.ShapeDtypeStruct((7 * _TOT[1],), jnp.float32),
        jax.ShapeDtypeStruct((7 * _TOT[2],), jnp.float32),
    ],
    scratch_types=[
        pltpu.VMEM((_TAB_W,), jnp.float32),
        pltpu.VMEM((_RPW * 8 + 8,), jnp.float32),
        pltpu.VMEM((_N,), jnp.float32),
        pltpu.VMEM((16 + 16,), jnp.int32),
        pltpu.VMEM((32 + 16,), jnp.int32),
        pltpu.VMEM((64 + 16,), jnp.int32),
        pltpu.VMEM((7 * _GRP * 16,), jnp.float32),
        pltpu.VMEM((7 * _GRP * 32,), jnp.float32),
        pltpu.VMEM((7 * _GRP * 64,), jnp.float32),
        pltpu.SMEM((8,), jnp.int32),
    ],
)(_sc_body)


def _run_grouper(xyz, feature, new_xyz):
    feat_t = jnp.transpose(feature, (0, 2, 1))
    tab = jnp.concatenate([xyz, feat_t], axis=-1).reshape(-1)
    cen8 = jnp.zeros((_B * _NPOINTS, 8), jnp.float32)
    cen8 = cen8.at[:, :3].set(new_xyz.reshape(_B * _NPOINTS, 3))
    cen = cen8.reshape(-1)
    sq = _run_sq(new_xyz, xyz).reshape(-1)
    g1, g2, g3 = _sc_grouper(tab, cen, sq)
    xs = []
    for g, ns in zip((g1, g2, g3), _NSAMPLES):
        xs.append(g.reshape(7, _B, _NPOINTS, ns).transpose(1, 0, 2, 3))
    return xs


def _square_distance(src, dst):
    return (jnp.sum(src ** 2, -1)[:, :, None] + jnp.sum(dst ** 2, -1)[:, None, :]
            - 2.0 * jnp.einsum('bmd,bnd->bmn', src, dst))


def _ball_query(radius, nsample, xyz, new_xyz):
    b, n, _ = xyz.shape
    m = new_xyz.shape[1]
    sqr = _square_distance(new_xyz, xyz)
    gidx = jnp.broadcast_to(jnp.arange(n, dtype=jnp.int32), (b, m, n))
    gidx = jnp.where(sqr > radius ** 2, n, gidx)
    gidx = jnp.sort(gidx, axis=-1)[:, :, :nsample]
    first = gidx[:, :, :1]
    first = jnp.where(first == n, 0, first)
    gidx = jnp.where(gidx == n, first, gidx)
    return gidx


def _gather_points(points, idx):
    bsz = points.shape[0]
    bidx = jnp.arange(bsz).reshape((bsz,) + (1,) * (idx.ndim - 1))
    return points[bidx, idx]


def _batchnorm(x, gamma, beta, axes):
    mean = jnp.mean(x, axis=axes, keepdims=True)
    var = jnp.var(x, axis=axes, keepdims=True)
    xh = (x - mean) * jax.lax.rsqrt(var + 1e-5)
    shape = [1] * x.ndim
    shape[1] = x.shape[1]
    return xh * gamma.reshape(shape) + beta.reshape(shape)


def kernel(xyz, feature, mlp_params, conv1_W, conv1_b, bn1_gamma, bn1_beta, fps_idx):
    inds, new_xyz = _run_fps(xyz)
    xs = _run_grouper(xyz, feature, new_xyz)
    outs = []
    for i in range(len(_RADII)):
        x = xs[i]
        for (W, bb, gm, bt) in mlp_params[i]:
            x = jnp.einsum('oi,bims->boms', W, x) + bb[None, :, None, None]
            x = jax.nn.relu(_batchnorm(x, gm, bt, (0, 2, 3)))
        outs.append(jnp.max(x, axis=-1))
    nf = jnp.concatenate(outs, axis=1)
    nf = jnp.einsum('oi,bim->bom', conv1_W, nf) + conv1_b[None, :, None]
    nf = jax.nn.relu(_batchnorm(nf, bn1_gamma, bn1_beta, (0, 2)))
    return new_xyz, nf, inds
